# baseline (speedup 1.0000x reference)
import functools

import jax
import jax.numpy as jnp
from jax import lax
from jax.experimental import pallas as pl
from jax.experimental.pallas import tpu as pltpu
from jax.experimental.pallas import tpu_sc as plsc

NC = 2
NS = 16
NW = NC * NS
LANES = 16

DS = 4
PITCH = 129


def _positional_encoding(seq_len: int, d_model: int) -> jax.Array:
    pos = jnp.arange(seq_len, dtype=jnp.float32)[:, None]
    _2i = jnp.arange(0, d_model, 2, dtype=jnp.float32)
    enc = jnp.zeros((seq_len, d_model), dtype=jnp.float32)
    enc = enc.at[:, 0::2].set(jnp.sin(pos / (10000.0 ** (_2i / d_model))))
    enc = enc.at[:, 1::2].set(jnp.cos(pos / (10000.0 ** (_2i / d_model))))
    return enc


@functools.partial(jax.jit, static_argnames=("B", "S", "D"))
def _embed_sc(xT, table, enc, *, B, S, D):
    BW = B // NW
    G = S // DS
    NV = D // LANES

    mesh = plsc.VectorSubcoreMesh(core_axis_name="c", subcore_axis_name="s")

    @functools.partial(
        pl.kernel,
        mesh=mesh,
        compiler_params=pltpu.CompilerParams(
            use_tc_tiling_on_sc=False, needs_layout_passes=False),
        out_type=jax.ShapeDtypeStruct((S, D, B), jnp.float32),
        scratch_types=[
            pltpu.VMEM((DS, BW), jnp.int32),
            pltpu.VMEM((DS * BW, D), jnp.float32),
            pltpu.VMEM((DS, D, PITCH), jnp.float32),
            pltpu.VMEM((S, D), jnp.float32),
            pltpu.SemaphoreType.DMA,
        ],
    )
    def body(xT_hbm, table_hbm, enc_hbm, out_hbm, idx_v, gbuf_v, obuf_v,
             enc_v, sem):
        wid = lax.axis_index("s") * NC + lax.axis_index("c")
        b0 = wid * BW
        pltpu.sync_copy(enc_hbm, enc_v)
        iota = lax.iota(jnp.int32, LANES)

        def chunk(g, carry):
            s0 = g * DS
            pltpu.sync_copy(xT_hbm.at[pl.ds(s0, DS), pl.ds(b0, BW)], idx_v)
            cps = [
                pltpu.async_copy(
                    table_hbm.at[idx_v.at[j]],
                    gbuf_v.at[pl.ds(j * BW, BW), :],
                    sem,
                )
                for j in range(DS)
            ]
            for cp in cps:
                cp.wait()

            def tr_row(b, c2):
                col = jnp.full((LANES,), b, jnp.int32)
                for j in range(DS):
                    r = j * BW + b
                    for d in range(NV):
                        e = enc_v[s0 + j, pl.ds(d * LANES, LANES)]
                        val = gbuf_v[r, pl.ds(d * LANES, LANES)] + e
                        plsc.store_scatter(
                            obuf_v,
                            [jnp.full((LANES,), j, jnp.int32),
                             iota + d * LANES, col],
                            val,
                        )
                return c2

            lax.fori_loop(0, BW, tr_row, 0)
            pltpu.sync_copy(
                obuf_v.at[:, :, pl.ds(0, BW)],
                out_hbm.at[pl.ds(s0, DS), :, pl.ds(b0, BW)],
            )
            return carry

        lax.fori_loop(0, G, chunk, 0)

    return body(xT, table, enc)


def kernel(x, table):
    B, S = x.shape
    _, D = table.shape
    xT = x.T
    enc = _positional_encoding(S, D)
    out3d = _embed_sc(xT, table, enc, B=B, S=S, D=D)
    return out3d.transpose(2, 0, 1)


# speedup vs baseline: 1.4155x; 1.4155x over previous
import functools

import jax
import jax.numpy as jnp
from jax import lax
from jax.experimental import pallas as pl
from jax.experimental.pallas import tpu as pltpu
from jax.experimental.pallas import tpu_sc as plsc

NC = 2
NS = 16
NW = NC * NS
LANES = 16

C_SEQ = 2
SUB = 100


def _positional_encoding(seq_len: int, d_model: int) -> jax.Array:
    pos = jnp.arange(seq_len, dtype=jnp.float32)[:, None]
    _2i = jnp.arange(0, d_model, 2, dtype=jnp.float32)
    enc = jnp.zeros((seq_len, d_model), dtype=jnp.float32)
    enc = enc.at[:, 0::2].set(jnp.sin(pos / (10000.0 ** (_2i / d_model))))
    enc = enc.at[:, 1::2].set(jnp.cos(pos / (10000.0 ** (_2i / d_model))))
    return enc


@functools.partial(jax.jit, static_argnames=("B", "S", "D"))
def _embed_sc(idx2d, table, enc, *, B, S, D):
    N = B * S
    R = C_SEQ * S
    KSUB = R // SUB
    rows_per_w = N // NW
    seqs_per_w = rows_per_w // S
    G = rows_per_w // R
    srows_per_w = rows_per_w // SUB

    mesh = plsc.VectorSubcoreMesh(core_axis_name="c", subcore_axis_name="s")

    @functools.partial(
        pl.kernel,
        mesh=mesh,
        compiler_params=pltpu.CompilerParams(use_tc_tiling_on_sc=False),
        out_type=jax.ShapeDtypeStruct((B, S, D), jnp.float32),
        scratch_types=[
            pltpu.VMEM((2, KSUB, SUB), jnp.int32),
            pltpu.VMEM((2, R, D), jnp.float32),
            pltpu.VMEM((2, C_SEQ, S, D), jnp.float32),
            pltpu.VMEM((S, D), jnp.float32),
            pltpu.SemaphoreType.DMA,
            pltpu.SemaphoreType.DMA,
            pltpu.SemaphoreType.DMA,
            pltpu.SemaphoreType.DMA,
        ],
    )
    def body(idx_hbm, table_hbm, enc_hbm, out_hbm, idx_v, gbuf_v, obuf_v,
             enc_v, gsem0, gsem1, osem0, osem1):
        wid = lax.axis_index("s") * NC + lax.axis_index("c")
        pltpu.sync_copy(enc_hbm, enc_v)
        gsems = (gsem0, gsem1)
        osems = (osem0, osem1)

        def gather_copies(g, p):
            srow0 = wid * srows_per_w + g * KSUB
            return [
                pltpu.make_async_copy(
                    table_hbm.at[idx_v.at[p, k]],
                    gbuf_v.at[p, pl.ds(k * SUB, SUB), :],
                    gsems[p],
                )
                for k in range(KSUB)
            ], srow0

        def start_chunk(g, p):
            cps, srow0 = gather_copies(g, p)
            pltpu.sync_copy(idx_hbm.at[pl.ds(srow0, KSUB), :], idx_v.at[p])
            for cp in cps:
                cp.start()

        def out_copy(g, p):
            b0 = wid * seqs_per_w + g * C_SEQ
            return pltpu.make_async_copy(
                obuf_v.at[p],
                out_hbm.at[pl.ds(b0, C_SEQ)],
                osems[p],
            )

        def process(g, p):
            gn = g + 1

            @pl.when(gn < G)
            def _():
                start_chunk(gn, 1 - p)

            cps, _ = gather_copies(g, p)
            for cp in cps:
                cp.wait()

            @pl.when(g >= 2)
            def _():
                out_copy(g - 2, p).wait()

            def add_row(s, c2):
                for d in range(D // LANES):
                    sl = pl.ds(d * LANES, LANES)
                    e = enc_v[s, sl]
                    for c in range(C_SEQ):
                        obuf_v[p, c, s, sl] = gbuf_v[p, c * S + s, sl] + e
                return c2

            lax.fori_loop(0, S, add_row, 0)
            out_copy(g, p).start()

        start_chunk(0, 0)

        def step(g2, carry):
            process(2 * g2, 0)
            process(2 * g2 + 1, 1)
            return carry

        lax.fori_loop(0, G // 2, step, 0)
        out_copy(G - 2, 0).wait()
        out_copy(G - 1, 1).wait()

    return body(idx2d, table, enc)


def kernel(x, table):
    B, S = x.shape
    _, D = table.shape
    idx2d = x.reshape(B * S // SUB, SUB)
    enc = _positional_encoding(S, D)
    return _embed_sc(idx2d, table, enc, B=B, S=S, D=D)
